# Initial kernel scaffold; baseline (speedup 1.0000x reference)
import jax
import jax.numpy as jnp
from jax.experimental import pallas as pl


def kernel(rotMat, output_rotmats):
    raise NotImplementedError("write your pallas kernel here")


# speedup vs baseline: 1.3021x; 1.3021x over previous
import functools

import jax
import jax.numpy as jnp
from jax.experimental import pallas as pl
from jax.experimental.pallas import tpu as pltpu
from jax.experimental.pallas import tpu_sc as plsc

_BN = 2048


def _score_body(q_ref, g_ref, max_ref, idx_ref, *, bn, a_total):
    i = pl.program_id(0)
    prod = jnp.dot(q_ref[...], g_ref[...], preferred_element_type=jnp.float32)
    bmax = jnp.max(prod, axis=1, keepdims=True)
    col = jax.lax.broadcasted_iota(jnp.int32, prod.shape, 1)
    masked = jnp.where(prod == bmax, col, a_total)
    bidx = jnp.min(masked, axis=1, keepdims=True) + i * bn

    @pl.when(i == 0)
    def _():
        max_ref[...] = bmax
        idx_ref[...] = bidx

    @pl.when(i != 0)
    def _():
        better = bmax > max_ref[...]
        idx_ref[...] = jnp.where(better, bidx, idx_ref[...])
        max_ref[...] = jnp.where(better, bmax, max_ref[...])


def _score(q, gt):
    b, k = q.shape
    a = gt.shape[1]
    nblocks = a // _BN
    return pl.pallas_call(
        functools.partial(_score_body, bn=_BN, a_total=a),
        grid=(nblocks,),
        in_specs=[
            pl.BlockSpec((b, k), lambda i: (0, 0)),
            pl.BlockSpec((k, _BN), lambda i: (0, i)),
        ],
        out_specs=[
            pl.BlockSpec((b, 1), lambda i: (0, 0)),
            pl.BlockSpec((b, 1), lambda i: (0, 0)),
        ],
        out_shape=[
            jax.ShapeDtypeStruct((b, 1), jnp.float32),
            jax.ShapeDtypeStruct((b, 1), jnp.int32),
        ],
    )(q, gt)


def _sc_gather(table, idxs):
    n = idxs.shape[0]
    window = 128
    mesh = plsc.VectorSubcoreMesh(
        core_axis_name="core", subcore_axis_name="subcore"
    )
    idxs2 = idxs.reshape(1, n)
    out_type = jax.ShapeDtypeStruct((n, table.shape[1]), table.dtype)

    @functools.partial(pl.kernel, out_type=out_type, mesh=mesh)
    def run(x_hbm, i_hbm, o_hbm):
        def body(i_vmem, o_vmem):
            pltpu.sync_copy(x_hbm.at[i_vmem.at[0]], o_vmem)

        pltpu.emit_pipeline(
            body,
            grid=(n // window,),
            in_specs=[pl.BlockSpec((1, window), index_map=lambda i: (0, i))],
            out_specs=[
                pl.BlockSpec((window, table.shape[1]), index_map=lambda i: (i, 0))
            ],
            core_axis_name="subcore",
            dimension_semantics=(pltpu.PARALLEL,),
        )(i_hbm, o_hbm)

    return run(table, idxs2)


def kernel(rotMat, output_rotmats):
    b = rotMat.shape[0]
    a = output_rotmats.shape[0]
    q = rotMat.reshape(b, 9)
    g = output_rotmats.reshape(a, 9)
    qp = jnp.pad(q, ((0, 0), (0, 7)))
    gp = jnp.pad(g, ((0, 0), (0, 119)))
    gt = jnp.pad(g, ((0, 0), (0, 7))).T
    maxv, idxv = _score(qp, gt)
    dot_trace = maxv.reshape(b)
    idxs = idxv.reshape(b)
    nearest = _sc_gather(gp, idxs)[:, :9].reshape(b, 3, 3)
    return dot_trace, nearest
